# initial kernel scaffold (unmeasured)
import jax
import jax.numpy as jnp
from jax import lax
from jax.experimental import pallas as pl
from jax.experimental.pallas import tpu as pltpu

B, H, D, BS = 8, 8, 128, 16
NB = 512
NPAGES = 512
PAGES_PER_BLK = 64
NBLK = NPAGES // PAGES_PER_BLK
KBLK = PAGES_PER_BLK * BS
SCALE = D ** -0.5
NEG_INF = -1e30


def kernel(Q, K, V, bt, lens):
    def body(Q_ref, K_ref, V_ref, bt_ref, lens_ref, out_ref,
             m_ref, l_ref, acc_ref, comm_ref, send_sem, recv_sem):
        blk = pl.program_id(0)
        my_x = lax.axis_index("x")
        my_y = lax.axis_index("y")
        my_z = lax.axis_index("z")

        @pl.when(blk == 0)
        def _():
            m_ref[...] = jnp.full((H, B), NEG_INF, jnp.float32)
            l_ref[...] = jnp.zeros((H, B), jnp.float32)
            acc_ref[...] = jnp.zeros((H, B, D), jnp.float32)

        gid0 = my_x * NPAGES + blk * PAGES_PER_BLK
        gids = gid0 + lax.broadcasted_iota(jnp.int32, (1, 1, PAGES_PER_BLK), 2)
        pos = lax.broadcasted_iota(jnp.int32, (B, NB), 1)
        validmask = pos < lens_ref[...]
        eq = (bt_ref[...][:, :, None] == gids) & validmask[:, :, None]
        cnt = jnp.sum(eq.astype(jnp.float32), axis=1)
        crep = jnp.broadcast_to(
            cnt[:, :, None], (B, PAGES_PER_BLK, BS)
        ).reshape(B, KBLK)

        Qb = Q_ref[...].reshape(B, H, D).astype(jnp.bfloat16).transpose(1, 0, 2)
        Kb = K_ref[...].reshape(KBLK, H, D).astype(jnp.bfloat16).transpose(1, 0, 2)
        Vb = V_ref[...].reshape(KBLK, H, D).astype(jnp.bfloat16).transpose(1, 0, 2)

        S = lax.dot_general(
            Qb, Kb, (((2,), (2,)), ((0,), (0,))),
            preferred_element_type=jnp.float32,
        ) * SCALE
        smask = jnp.where(crep[None] > 0, S, NEG_INF)

        m_old = m_ref[...]
        m_new = jnp.maximum(m_old, jnp.max(smask, axis=2))
        alpha = jnp.exp(m_old - m_new)
        p = jnp.exp(smask - m_new[:, :, None]) * crep[None]
        l_ref[...] = l_ref[...] * alpha + jnp.sum(p, axis=2)
        o_blk = lax.dot_general(
            p.astype(jnp.bfloat16), Vb, (((2,), (1,)), ((0,), (0,))),
            preferred_element_type=jnp.float32,
        )
        acc_ref[...] = acc_ref[...] * alpha[:, :, None] + o_blk
        m_ref[...] = m_new

        @pl.when(blk == NBLK - 1)
        def _():
            comm_ref[0, :H, :, :] = acc_ref[...]
            comm_ref[0, H, :, 0:H] = m_ref[...].T
            comm_ref[0, H, :, H:2 * H] = l_ref[...].T
            rdma = pltpu.make_async_remote_copy(
                src_ref=comm_ref.at[0],
                dst_ref=comm_ref.at[1],
                send_sem=send_sem,
                recv_sem=recv_sem,
                device_id=(1 - my_x, my_y, my_z),
                device_id_type=pl.DeviceIdType.MESH,
            )
            rdma.start()
            rdma.wait()

            acc_o = comm_ref[1, :H, :, :]
            m_o = comm_ref[1, H, :, 0:H].T
            l_o = comm_ref[1, H, :, H:2 * H].T

            m_g = jnp.maximum(m_ref[...], m_o)
            a = jnp.exp(m_ref[...] - m_g)
            b = jnp.exp(m_o - m_g)
            l_g = l_ref[...] * a + l_o * b
            o_g = acc_ref[...] * a[:, :, None] + acc_o * b[:, :, None]
            res = o_g / l_g[:, :, None]
            out_ref[...] = res.transpose(1, 0, 2).reshape(B, 1, H, D)

    lens2 = lens.reshape(B, 1)
    return pl.pallas_call(
        body,
        grid=(NBLK,),
        in_specs=[
            pl.BlockSpec((B, 1, H, D), lambda b: (0, 0, 0, 0)),
            pl.BlockSpec((PAGES_PER_BLK, BS, H, D), lambda b: (b, 0, 0, 0)),
            pl.BlockSpec((PAGES_PER_BLK, BS, H, D), lambda b: (b, 0, 0, 0)),
            pl.BlockSpec((B, NB), lambda b: (0, 0)),
            pl.BlockSpec((B, 1), lambda b: (0, 0)),
        ],
        out_specs=pl.BlockSpec((B, 1, H, D), lambda b: (0, 0, 0, 0)),
        out_shape=jax.ShapeDtypeStruct((B, 1, H, D), jnp.float32),
        scratch_shapes=[
            pltpu.VMEM((H, B), jnp.float32),
            pltpu.VMEM((H, B), jnp.float32),
            pltpu.VMEM((H, B, D), jnp.float32),
            pltpu.VMEM((2, H + 1, B, D), jnp.float32),
            pltpu.SemaphoreType.DMA,
            pltpu.SemaphoreType.DMA,
        ],
        compiler_params=pltpu.CompilerParams(
            dimension_semantics=("arbitrary",),
            collective_id=0,
        ),
    )(Q, K, V, bt, lens2)


# baseline (device time: 90076 ns/iter reference)
import jax
import jax.numpy as jnp
from jax import lax
from jax.experimental import pallas as pl
from jax.experimental.pallas import tpu as pltpu

B, H, D, BS = 8, 8, 128, 16
NB = 512
NPAGES = 512
PAGES_PER_BLK = 64
NBLK = NPAGES // PAGES_PER_BLK
KBLK = PAGES_PER_BLK * BS
SCALE = D ** -0.5
NEG_INF = -1e30


def kernel(Q, K, V, bt, lens):
    Q2 = Q.reshape(B, H * D)
    K2 = K.reshape(NPAGES * BS, H * D)
    V2 = V.reshape(NPAGES * BS, H * D)
    lens2 = lens.reshape(B, 1)

    def body(Q_ref, K_ref, V_ref, bt_ref, lens_ref, out_ref,
             m_ref, l_ref, acc_ref, acc_rx, m_rx, l_rx,
             send_sems, recv_sems):
        blk = pl.program_id(0)
        my_x = lax.axis_index("x")
        my_y = lax.axis_index("y")
        my_z = lax.axis_index("z")

        @pl.when(blk == 0)
        def _():
            m_ref[...] = jnp.full((H, B, 1), NEG_INF, jnp.float32)
            l_ref[...] = jnp.zeros((H, B, 1), jnp.float32)
            acc_ref[...] = jnp.zeros((H, B, D), jnp.float32)

        gid0 = my_x * NPAGES + blk * PAGES_PER_BLK
        gids3 = gid0 + lax.broadcasted_iota(
            jnp.int32, (PAGES_PER_BLK, B, NB), 0)
        pos = lax.broadcasted_iota(jnp.int32, (B, NB), 1)
        valid = pos < lens_ref[...]
        eq = (bt_ref[...][None] == gids3) & valid[None]
        cnt_pi = jnp.sum(eq.astype(jnp.float32), axis=2)
        rowid = lax.broadcasted_iota(jnp.int32, (PAGES_PER_BLK, KBLK), 0)
        colpg = lax.broadcasted_iota(jnp.int32, (PAGES_PER_BLK, KBLK), 1) // BS
        onehot = (rowid == colpg).astype(jnp.float32)
        crep = lax.dot_general(
            cnt_pi, onehot, (((0,), (0,)), ((), ())),
            preferred_element_type=jnp.float32,
        )
        cpos = crep > 0

        for h in range(H):
            q_h = Q_ref[:, h * D:(h + 1) * D].astype(jnp.bfloat16)
            k_h = K_ref[:, h * D:(h + 1) * D].astype(jnp.bfloat16)
            v_h = V_ref[:, h * D:(h + 1) * D].astype(jnp.bfloat16)
            s_h = lax.dot_general(
                q_h, k_h, (((1,), (1,)), ((), ())),
                preferred_element_type=jnp.float32,
            ) * SCALE
            s_h = jnp.where(cpos, s_h, NEG_INF)
            m_old = m_ref[h]
            m_new = jnp.maximum(m_old, jnp.max(s_h, axis=1, keepdims=True))
            alpha = jnp.exp(m_old - m_new)
            p_h = jnp.exp(s_h - m_new) * crep
            l_ref[h] = l_ref[h] * alpha + jnp.sum(p_h, axis=1, keepdims=True)
            o_h = lax.dot_general(
                p_h.astype(jnp.bfloat16), v_h, (((1,), (0,)), ((), ())),
                preferred_element_type=jnp.float32,
            )
            acc_ref[h] = acc_ref[h] * alpha + o_h
            m_ref[h] = m_new

        @pl.when(blk == NBLK - 1)
        def _():
            peer = (1 - my_x, my_y, my_z)
            rds = []
            for i, (src, dst) in enumerate(
                [(acc_ref, acc_rx), (m_ref, m_rx), (l_ref, l_rx)]
            ):
                rds.append(pltpu.make_async_remote_copy(
                    src_ref=src, dst_ref=dst,
                    send_sem=send_sems.at[i], recv_sem=recv_sems.at[i],
                    device_id=peer, device_id_type=pl.DeviceIdType.MESH,
                ))
            for rd in rds:
                rd.start()
            for rd in rds:
                rd.wait()

            m_g = jnp.maximum(m_ref[...], m_rx[...])
            a = jnp.exp(m_ref[...] - m_g)
            b = jnp.exp(m_rx[...] - m_g)
            l_g = l_ref[...] * a + l_rx[...] * b
            o_g = acc_ref[...] * a + acc_rx[...] * b
            out_ref[...] = o_g / l_g

    out = pl.pallas_call(
        body,
        grid=(NBLK,),
        in_specs=[
            pl.BlockSpec((B, H * D), lambda b: (0, 0)),
            pl.BlockSpec((KBLK, H * D), lambda b: (b, 0)),
            pl.BlockSpec((KBLK, H * D), lambda b: (b, 0)),
            pl.BlockSpec((B, NB), lambda b: (0, 0)),
            pl.BlockSpec((B, 1), lambda b: (0, 0)),
        ],
        out_specs=pl.BlockSpec((H, B, D), lambda b: (0, 0, 0)),
        out_shape=jax.ShapeDtypeStruct((H, B, D), jnp.float32),
        scratch_shapes=[
            pltpu.VMEM((H, B, 1), jnp.float32),
            pltpu.VMEM((H, B, 1), jnp.float32),
            pltpu.VMEM((H, B, D), jnp.float32),
            pltpu.VMEM((H, B, D), jnp.float32),
            pltpu.VMEM((H, B, 1), jnp.float32),
            pltpu.VMEM((H, B, 1), jnp.float32),
            pltpu.SemaphoreType.DMA((3,)),
            pltpu.SemaphoreType.DMA((3,)),
        ],
        compiler_params=pltpu.CompilerParams(
            dimension_semantics=("arbitrary",),
        ),
    )(Q2, K2, V2, bt, lens2)
    return out.transpose(1, 0, 2).reshape(B, 1, H, D)


# device time: 44256 ns/iter; 2.0353x vs baseline; 2.0353x over previous
import jax
import jax.numpy as jnp
from jax import lax
from jax.experimental import pallas as pl
from jax.experimental.pallas import tpu as pltpu

B, H, D, BS = 8, 8, 128, 16
NB = 512
NPAGES = 512
PAGES_PER_BLK = 64
NBLK = NPAGES // PAGES_PER_BLK
KBLK = PAGES_PER_BLK * BS
SCALE = D ** -0.5
NEG_INF = -1e30


def kernel(Q, K, V, bt, lens):
    Q2 = Q.reshape(B, H * D)
    K3 = K.reshape(NPAGES * BS, H, D)
    V3 = V.reshape(NPAGES * BS, H, D)
    lens2 = lens.reshape(B, 1)

    def body(Q_ref, K_ref, V_ref, bt_ref, lens_ref, out_ref,
             m_ref, l_ref, acc_ref, kbuf, vbuf, acc_rx, m_rx, l_rx,
             ksem, vsem, send_sems, recv_sems):
        blk = pl.program_id(0)
        my_x = lax.axis_index("x")
        my_y = lax.axis_index("y")
        my_z = lax.axis_index("z")

        def block_copies(b, slot):
            cps = []
            for src_ref, buf, sem in ((K_ref, kbuf, ksem), (V_ref, vbuf, vsem)):
                for hh in range(H):
                    cps.append(pltpu.make_async_copy(
                        src_ref.at[pl.ds(b * KBLK, KBLK), hh, :],
                        buf.at[slot, hh],
                        sem.at[slot],
                    ))
            return cps

        @pl.when(blk == 0)
        def _():
            m_ref[...] = jnp.full((H, B, 1), NEG_INF, jnp.float32)
            l_ref[...] = jnp.zeros((H, B, 1), jnp.float32)
            acc_ref[...] = jnp.zeros((H, B, D), jnp.float32)
            for cp in block_copies(0, 0):
                cp.start()

        @pl.when(blk + 1 < NBLK)
        def _():
            for cp in block_copies(blk + 1, (blk + 1) % 2):
                cp.start()

        slot = blk % 2
        for cp in block_copies(blk, slot):
            cp.wait()

        gid0 = my_x * NPAGES + blk * PAGES_PER_BLK
        gids3 = gid0 + lax.broadcasted_iota(
            jnp.int32, (PAGES_PER_BLK, B, NB), 0)
        pos = lax.broadcasted_iota(jnp.int32, (B, NB), 1)
        valid = pos < lens_ref[...]
        eq = (bt_ref[...][None] == gids3) & valid[None]
        cnt_pi = jnp.sum(eq.astype(jnp.float32), axis=2)
        rowid = lax.broadcasted_iota(jnp.int32, (PAGES_PER_BLK, KBLK), 0)
        colpg = lax.broadcasted_iota(
            jnp.int32, (PAGES_PER_BLK, KBLK), 1) // BS
        onehot = (rowid == colpg).astype(jnp.float32)
        crep = lax.dot_general(
            cnt_pi, onehot, (((0,), (0,)), ((), ())),
            preferred_element_type=jnp.float32,
        )
        cpos = crep > 0

        for h in range(H):
            q_h = Q_ref[:, h * D:(h + 1) * D].astype(jnp.bfloat16)
            k_h = kbuf[slot, h].astype(jnp.bfloat16)
            v_h = vbuf[slot, h].astype(jnp.bfloat16)
            s_h = lax.dot_general(
                q_h, k_h, (((1,), (1,)), ((), ())),
                preferred_element_type=jnp.float32,
            ) * SCALE
            s_h = jnp.where(cpos, s_h, NEG_INF)
            m_old = m_ref[h]
            m_new = jnp.maximum(m_old, jnp.max(s_h, axis=1, keepdims=True))
            alpha = jnp.exp(m_old - m_new)
            p_h = jnp.exp(s_h - m_new) * crep
            l_ref[h] = l_ref[h] * alpha + jnp.sum(p_h, axis=1, keepdims=True)
            o_h = lax.dot_general(
                p_h.astype(jnp.bfloat16), v_h, (((1,), (0,)), ((), ())),
                preferred_element_type=jnp.float32,
            )
            acc_ref[h] = acc_ref[h] * alpha + o_h
            m_ref[h] = m_new

        @pl.when(blk == NBLK - 1)
        def _():
            peer = (1 - my_x, my_y, my_z)
            rds = []
            for i, (src, dst) in enumerate(
                [(acc_ref, acc_rx), (m_ref, m_rx), (l_ref, l_rx)]
            ):
                rds.append(pltpu.make_async_remote_copy(
                    src_ref=src, dst_ref=dst,
                    send_sem=send_sems.at[i], recv_sem=recv_sems.at[i],
                    device_id=peer, device_id_type=pl.DeviceIdType.MESH,
                ))
            for rd in rds:
                rd.start()
            for rd in rds:
                rd.wait()

            m_g = jnp.maximum(m_ref[...], m_rx[...])
            a = jnp.exp(m_ref[...] - m_g)
            b = jnp.exp(m_rx[...] - m_g)
            l_g = l_ref[...] * a + l_rx[...] * b
            o_g = acc_ref[...] * a + acc_rx[...] * b
            out_ref[...] = o_g / l_g

    out = pl.pallas_call(
        body,
        grid=(NBLK,),
        in_specs=[
            pl.BlockSpec((B, H * D), lambda b: (0, 0)),
            pl.BlockSpec(memory_space=pltpu.MemorySpace.HBM),
            pl.BlockSpec(memory_space=pltpu.MemorySpace.HBM),
            pl.BlockSpec((B, NB), lambda b: (0, 0)),
            pl.BlockSpec((B, 1), lambda b: (0, 0)),
        ],
        out_specs=pl.BlockSpec((H, B, D), lambda b: (0, 0, 0)),
        out_shape=jax.ShapeDtypeStruct((H, B, D), jnp.float32),
        scratch_shapes=[
            pltpu.VMEM((H, B, 1), jnp.float32),
            pltpu.VMEM((H, B, 1), jnp.float32),
            pltpu.VMEM((H, B, D), jnp.float32),
            pltpu.VMEM((2, H, KBLK, D), jnp.float32),
            pltpu.VMEM((2, H, KBLK, D), jnp.float32),
            pltpu.VMEM((H, B, D), jnp.float32),
            pltpu.VMEM((H, B, 1), jnp.float32),
            pltpu.VMEM((H, B, 1), jnp.float32),
            pltpu.SemaphoreType.DMA((2,)),
            pltpu.SemaphoreType.DMA((2,)),
            pltpu.SemaphoreType.DMA((3,)),
            pltpu.SemaphoreType.DMA((3,)),
        ],
        compiler_params=pltpu.CompilerParams(
            dimension_semantics=("arbitrary",),
        ),
    )(Q2, K3, V3, bt, lens2)
    return out.transpose(1, 0, 2).reshape(B, 1, H, D)


# device time: 29785 ns/iter; 3.0242x vs baseline; 1.4858x over previous
import jax
import jax.numpy as jnp
from jax import lax
from jax.experimental import pallas as pl
from jax.experimental.pallas import tpu as pltpu

B, H, D, BS = 8, 8, 128, 16
NB = 512
NPAGES = 512
QPAGES = 128
PAGES_PER_BLK = 64
NBLK = QPAGES // PAGES_PER_BLK
KBLK = PAGES_PER_BLK * BS
SCALE = D ** -0.5
NEG_INF = -1e30


def kernel(Q, K, V, bt, lens):
    Q2 = Q.reshape(B, H * D)
    K3 = K.reshape(NPAGES * BS, H, D)
    V3 = V.reshape(NPAGES * BS, H, D)
    lens2 = lens.reshape(B, 1)

    def body(Q_ref, K_ref, V_ref, bt_ref, lens_ref, out_ref,
             m_ref, l_ref, acc_ref, kbuf, vbuf, acc_rx, m_rx, l_rx,
             ksem, vsem, send_sems, recv_sems):
        blk = pl.program_id(0)
        my_x = lax.axis_index("x")
        my_y = lax.axis_index("y")
        my_z = lax.axis_index("z")
        quarter = 2 * my_y + my_z

        def block_copies(b, slot):
            row0 = (quarter * QPAGES + b * PAGES_PER_BLK) * BS
            cps = []
            for src_ref, buf, sem in ((K_ref, kbuf, ksem), (V_ref, vbuf, vsem)):
                for hh in range(H):
                    cps.append(pltpu.make_async_copy(
                        src_ref.at[pl.ds(row0, KBLK), hh, :],
                        buf.at[slot, hh],
                        sem.at[slot],
                    ))
            return cps

        @pl.when(blk == 0)
        def _():
            m_ref[...] = jnp.full((H, B, 1), NEG_INF, jnp.float32)
            l_ref[...] = jnp.zeros((H, B, 1), jnp.float32)
            acc_ref[...] = jnp.zeros((H, B, D), jnp.float32)
            for cp in block_copies(0, 0):
                cp.start()

        @pl.when(blk + 1 < NBLK)
        def _():
            for cp in block_copies(blk + 1, (blk + 1) % 2):
                cp.start()

        slot = blk % 2
        for cp in block_copies(blk, slot):
            cp.wait()

        gid0 = my_x * NPAGES + quarter * QPAGES + blk * PAGES_PER_BLK
        gids3 = gid0 + lax.broadcasted_iota(
            jnp.int32, (PAGES_PER_BLK, B, NB), 0)
        pos = lax.broadcasted_iota(jnp.int32, (B, NB), 1)
        valid = pos < lens_ref[...]
        eq = (bt_ref[...][None] == gids3) & valid[None]
        cnt_pi = jnp.sum(eq.astype(jnp.float32), axis=2)
        rowid = lax.broadcasted_iota(jnp.int32, (PAGES_PER_BLK, KBLK), 0)
        colpg = lax.broadcasted_iota(
            jnp.int32, (PAGES_PER_BLK, KBLK), 1) // BS
        onehot = (rowid == colpg).astype(jnp.float32)
        crep = lax.dot_general(
            cnt_pi, onehot, (((0,), (0,)), ((), ())),
            preferred_element_type=jnp.float32,
        )
        cpos = crep > 0

        for h in range(H):
            q_h = Q_ref[:, h * D:(h + 1) * D].astype(jnp.bfloat16)
            k_h = kbuf[slot, h].astype(jnp.bfloat16)
            v_h = vbuf[slot, h].astype(jnp.bfloat16)
            s_h = lax.dot_general(
                q_h, k_h, (((1,), (1,)), ((), ())),
                preferred_element_type=jnp.float32,
            ) * SCALE
            s_h = jnp.where(cpos, s_h, NEG_INF)
            m_old = m_ref[h]
            m_new = jnp.maximum(m_old, jnp.max(s_h, axis=1, keepdims=True))
            alpha = jnp.exp(m_old - m_new)
            p_h = jnp.exp(s_h - m_new) * crep
            l_ref[h] = l_ref[h] * alpha + jnp.sum(p_h, axis=1, keepdims=True)
            o_h = lax.dot_general(
                p_h.astype(jnp.bfloat16), v_h, (((1,), (0,)), ((), ())),
                preferred_element_type=jnp.float32,
            )
            acc_ref[h] = acc_ref[h] * alpha + o_h
            m_ref[h] = m_new

        @pl.when(blk == NBLK - 1)
        def _():
            peers = [
                (my_x, my_y, 1 - my_z),
                (my_x, 1 - my_y, my_z),
                (1 - my_x, my_y, my_z),
            ]
            for stage, peer in enumerate(peers):
                rds = []
                for i, (src, dst) in enumerate(
                    [(acc_ref, acc_rx), (m_ref, m_rx), (l_ref, l_rx)]
                ):
                    rds.append(pltpu.make_async_remote_copy(
                        src_ref=src, dst_ref=dst.at[stage],
                        send_sem=send_sems.at[3 * stage + i],
                        recv_sem=recv_sems.at[3 * stage + i],
                        device_id=peer, device_id_type=pl.DeviceIdType.MESH,
                    ))
                for rd in rds:
                    rd.start()
                for rd in rds:
                    rd.wait()

                m_g = jnp.maximum(m_ref[...], m_rx[stage])
                a = jnp.exp(m_ref[...] - m_g)
                b = jnp.exp(m_rx[stage] - m_g)
                l_ref[...] = l_ref[...] * a + l_rx[stage] * b
                acc_ref[...] = acc_ref[...] * a + acc_rx[stage] * b
                m_ref[...] = m_g

            out_ref[...] = acc_ref[...] / l_ref[...]

    out = pl.pallas_call(
        body,
        grid=(NBLK,),
        in_specs=[
            pl.BlockSpec((B, H * D), lambda b: (0, 0)),
            pl.BlockSpec(memory_space=pltpu.MemorySpace.HBM),
            pl.BlockSpec(memory_space=pltpu.MemorySpace.HBM),
            pl.BlockSpec((B, NB), lambda b: (0, 0)),
            pl.BlockSpec((B, 1), lambda b: (0, 0)),
        ],
        out_specs=pl.BlockSpec((H, B, D), lambda b: (0, 0, 0)),
        out_shape=jax.ShapeDtypeStruct((H, B, D), jnp.float32),
        scratch_shapes=[
            pltpu.VMEM((H, B, 1), jnp.float32),
            pltpu.VMEM((H, B, 1), jnp.float32),
            pltpu.VMEM((H, B, D), jnp.float32),
            pltpu.VMEM((2, H, KBLK, D), jnp.float32),
            pltpu.VMEM((2, H, KBLK, D), jnp.float32),
            pltpu.VMEM((3, H, B, D), jnp.float32),
            pltpu.VMEM((3, H, B, 1), jnp.float32),
            pltpu.VMEM((3, H, B, 1), jnp.float32),
            pltpu.SemaphoreType.DMA((2,)),
            pltpu.SemaphoreType.DMA((2,)),
            pltpu.SemaphoreType.DMA((9,)),
            pltpu.SemaphoreType.DMA((9,)),
        ],
        compiler_params=pltpu.CompilerParams(
            dimension_semantics=("arbitrary",),
        ),
    )(Q2, K3, V3, bt, lens2)
    return out.transpose(1, 0, 2).reshape(B, 1, H, D)


# device time: 25155 ns/iter; 3.5808x vs baseline; 1.1841x over previous
import jax
import jax.numpy as jnp
from jax import lax
from jax.experimental import pallas as pl
from jax.experimental.pallas import tpu as pltpu

B, H, D, BS = 8, 8, 128, 16
NB = 512
NPAGES = 512
QPAGES = 128
PAGES_PER_BLK = 64
NBLK = QPAGES // PAGES_PER_BLK
KBLK = PAGES_PER_BLK * BS
SCALE = D ** -0.5
NEG_INF = -1e30


def kernel(Q, K, V, bt, lens):
    Q2 = Q.reshape(B, H * D)
    K3 = K.reshape(NPAGES * BS, H, D)
    V3 = V.reshape(NPAGES * BS, H, D)
    lens2 = lens.reshape(B, 1)

    def body(Q_ref, K_ref, V_ref, bt_ref, lens_ref, out_ref,
             m_ref, l_ref, acc_ref, kbuf, vbuf, acc_rx, m_rx, l_rx,
             ksem, vsem, send_sems, recv_sems):
        blk = pl.program_id(0)
        my_x = lax.axis_index("x")
        my_y = lax.axis_index("y")
        my_z = lax.axis_index("z")
        quarter = 2 * my_y + my_z

        def block_copies(b, slot):
            row0 = (quarter * QPAGES + b * PAGES_PER_BLK) * BS
            cps = []
            for src_ref, buf, sem in ((K_ref, kbuf, ksem), (V_ref, vbuf, vsem)):
                for hh in range(H):
                    cps.append(pltpu.make_async_copy(
                        src_ref.at[pl.ds(row0, KBLK), hh, :],
                        buf.at[slot, hh],
                        sem.at[slot],
                    ))
            return cps

        @pl.when(blk == 0)
        def _():
            m_ref[...] = jnp.full((H, B, 1), NEG_INF, jnp.float32)
            l_ref[...] = jnp.zeros((H, B, 1), jnp.float32)
            acc_ref[...] = jnp.zeros((H, B, D), jnp.float32)
            for cp in block_copies(0, 0):
                cp.start()

        @pl.when(blk + 1 < NBLK)
        def _():
            for cp in block_copies(blk + 1, (blk + 1) % 2):
                cp.start()

        @pl.when(blk == 0)
        def _():
            barrier_sem = pltpu.get_barrier_semaphore()
            for peer in [
                (my_x, my_y, 1 - my_z),
                (my_x, 1 - my_y, my_z),
                (1 - my_x, my_y, my_z),
            ]:
                pl.semaphore_signal(
                    barrier_sem, inc=1,
                    device_id=peer, device_id_type=pl.DeviceIdType.MESH,
                )
            pl.semaphore_wait(barrier_sem, 3)

        gid0 = my_x * NPAGES + quarter * QPAGES + blk * PAGES_PER_BLK
        gids3 = gid0 + lax.broadcasted_iota(
            jnp.int32, (PAGES_PER_BLK, B, NB), 0)
        pos = lax.broadcasted_iota(jnp.int32, (B, NB), 1)
        valid = pos < lens_ref[...]
        eq = (bt_ref[...][None] == gids3) & valid[None]
        cnt_pi = jnp.sum(eq.astype(jnp.float32), axis=2)
        rowid = lax.broadcasted_iota(jnp.int32, (PAGES_PER_BLK, KBLK), 0)
        colpg = lax.broadcasted_iota(
            jnp.int32, (PAGES_PER_BLK, KBLK), 1) // BS
        onehot = (rowid == colpg).astype(jnp.float32)
        crep = lax.dot_general(
            cnt_pi, onehot, (((0,), (0,)), ((), ())),
            preferred_element_type=jnp.float32,
        )
        cpos = crep > 0

        slot = blk % 2
        for cp in block_copies(blk, slot):
            cp.wait()

        for h in range(H):
            q_h = Q_ref[:, h * D:(h + 1) * D].astype(jnp.bfloat16)
            k_h = kbuf[slot, h].astype(jnp.bfloat16)
            v_h = vbuf[slot, h].astype(jnp.bfloat16)
            s_h = lax.dot_general(
                q_h, k_h, (((1,), (1,)), ((), ())),
                preferred_element_type=jnp.float32,
            ) * SCALE
            s_h = jnp.where(cpos, s_h, NEG_INF)
            m_old = m_ref[h]
            m_new = jnp.maximum(m_old, jnp.max(s_h, axis=1, keepdims=True))
            alpha = jnp.exp(m_old - m_new)
            p_h = jnp.exp(s_h - m_new) * crep
            l_ref[h] = l_ref[h] * alpha + jnp.sum(p_h, axis=1, keepdims=True)
            o_h = lax.dot_general(
                p_h.astype(jnp.bfloat16), v_h, (((1,), (0,)), ((), ())),
                preferred_element_type=jnp.float32,
            )
            acc_ref[h] = acc_ref[h] * alpha + o_h
            m_ref[h] = m_new

        @pl.when(blk == NBLK - 1)
        def _():
            peers = [
                (my_x, my_y, 1 - my_z),
                (my_x, 1 - my_y, my_z),
                (1 - my_x, my_y, my_z),
            ]
            for stage, peer in enumerate(peers):
                rds = []
                for i, (src, dst) in enumerate(
                    [(acc_ref, acc_rx), (m_ref, m_rx), (l_ref, l_rx)]
                ):
                    rds.append(pltpu.make_async_remote_copy(
                        src_ref=src, dst_ref=dst.at[stage],
                        send_sem=send_sems.at[3 * stage + i],
                        recv_sem=recv_sems.at[3 * stage + i],
                        device_id=peer, device_id_type=pl.DeviceIdType.MESH,
                    ))
                for rd in rds:
                    rd.start()
                for rd in rds:
                    rd.wait()

                m_g = jnp.maximum(m_ref[...], m_rx[stage])
                a = jnp.exp(m_ref[...] - m_g)
                b = jnp.exp(m_rx[stage] - m_g)
                l_ref[...] = l_ref[...] * a + l_rx[stage] * b
                acc_ref[...] = acc_ref[...] * a + acc_rx[stage] * b
                m_ref[...] = m_g

            out_ref[...] = acc_ref[...] / l_ref[...]

    out = pl.pallas_call(
        body,
        grid=(NBLK,),
        in_specs=[
            pl.BlockSpec((B, H * D), lambda b: (0, 0)),
            pl.BlockSpec(memory_space=pltpu.MemorySpace.HBM),
            pl.BlockSpec(memory_space=pltpu.MemorySpace.HBM),
            pl.BlockSpec((B, NB), lambda b: (0, 0)),
            pl.BlockSpec((B, 1), lambda b: (0, 0)),
        ],
        out_specs=pl.BlockSpec((H, B, D), lambda b: (0, 0, 0)),
        out_shape=jax.ShapeDtypeStruct((H, B, D), jnp.float32),
        scratch_shapes=[
            pltpu.VMEM((H, B, 1), jnp.float32),
            pltpu.VMEM((H, B, 1), jnp.float32),
            pltpu.VMEM((H, B, D), jnp.float32),
            pltpu.VMEM((2, H, KBLK, D), jnp.float32),
            pltpu.VMEM((2, H, KBLK, D), jnp.float32),
            pltpu.VMEM((3, H, B, D), jnp.float32),
            pltpu.VMEM((3, H, B, 1), jnp.float32),
            pltpu.VMEM((3, H, B, 1), jnp.float32),
            pltpu.SemaphoreType.DMA((2,)),
            pltpu.SemaphoreType.DMA((2,)),
            pltpu.SemaphoreType.DMA((9,)),
            pltpu.SemaphoreType.DMA((9,)),
        ],
        compiler_params=pltpu.CompilerParams(
            dimension_semantics=("arbitrary",),
            collective_id=0,
        ),
    )(Q2, K3, V3, bt, lens2)
    return out.transpose(1, 0, 2).reshape(B, 1, H, D)


# device time: 24299 ns/iter; 3.7070x vs baseline; 1.0352x over previous
import jax
import jax.numpy as jnp
from jax import lax
from jax.experimental import pallas as pl
from jax.experimental.pallas import tpu as pltpu

B, H, D, BS = 8, 8, 128, 16
NB = 512
NPAGES = 512
QPAGES = 128
PAGES_PER_BLK = 64
NBLK = QPAGES // PAGES_PER_BLK
KBLK = PAGES_PER_BLK * BS
SCALE = D ** -0.5
NEG_INF = -1e30


def kernel(Q, K, V, bt, lens):
    Q2 = Q.reshape(B, H * D)
    K3 = K.reshape(NPAGES * BS, H, D)
    V3 = V.reshape(NPAGES * BS, H, D)
    lens2 = lens.reshape(B, 1)

    def body(Q_ref, K_ref, V_ref, bt_ref, lens_ref, out_ref,
             m_ref, l_ref, acc_ref, kbuf, vbuf, sbuf, rbuf,
             ksem, vsem, send_sems, recv_sems):
        blk = pl.program_id(0)
        my_x = lax.axis_index("x")
        my_y = lax.axis_index("y")
        my_z = lax.axis_index("z")
        quarter = 2 * my_y + my_z

        def block_copies(b, slot):
            row0 = (quarter * QPAGES + b * PAGES_PER_BLK) * BS
            cps = []
            for src_ref, buf, sem in ((K_ref, kbuf, ksem), (V_ref, vbuf, vsem)):
                for hh in range(H):
                    cps.append(pltpu.make_async_copy(
                        src_ref.at[pl.ds(row0, KBLK), hh, :],
                        buf.at[slot, hh],
                        sem.at[slot],
                    ))
            return cps

        @pl.when(blk == 0)
        def _():
            m_ref[...] = jnp.full((H, B, 1), NEG_INF, jnp.float32)
            l_ref[...] = jnp.zeros((H, B, 1), jnp.float32)
            acc_ref[...] = jnp.zeros((H, B, D), jnp.float32)
            for cp in block_copies(0, 0):
                cp.start()

        @pl.when(blk + 1 < NBLK)
        def _():
            for cp in block_copies(blk + 1, (blk + 1) % 2):
                cp.start()

        @pl.when(blk == 0)
        def _():
            barrier_sem = pltpu.get_barrier_semaphore()
            for peer in [
                (my_x, my_y, 1 - my_z),
                (my_x, 1 - my_y, my_z),
                (1 - my_x, my_y, my_z),
            ]:
                pl.semaphore_signal(
                    barrier_sem, inc=1,
                    device_id=peer, device_id_type=pl.DeviceIdType.MESH,
                )
            pl.semaphore_wait(barrier_sem, 3)

        gid0 = my_x * NPAGES + quarter * QPAGES + blk * PAGES_PER_BLK
        gids3 = gid0 + lax.broadcasted_iota(
            jnp.int32, (PAGES_PER_BLK, B, NB), 0)
        pos = lax.broadcasted_iota(jnp.int32, (B, NB), 1)
        valid = pos < lens_ref[...]
        eq = (bt_ref[...][None] == gids3) & valid[None]
        cnt_pi = jnp.sum(eq.astype(jnp.float32), axis=2)
        rowid = lax.broadcasted_iota(jnp.int32, (PAGES_PER_BLK, KBLK), 0)
        colpg = lax.broadcasted_iota(
            jnp.int32, (PAGES_PER_BLK, KBLK), 1) // BS
        onehot = (rowid == colpg).astype(jnp.float32)
        crep = lax.dot_general(
            cnt_pi, onehot, (((0,), (0,)), ((), ())),
            preferred_element_type=jnp.float32,
        )
        cpos = crep > 0

        slot = blk % 2
        for cp in block_copies(blk, slot):
            cp.wait()

        for h in range(H):
            q_h = Q_ref[:, h * D:(h + 1) * D]
            k_h = kbuf[slot, h]
            v_h = vbuf[slot, h]
            s_h = lax.dot_general(
                q_h, k_h, (((1,), (1,)), ((), ())),
                preferred_element_type=jnp.float32,
            ) * SCALE
            s_h = jnp.where(cpos, s_h, NEG_INF)
            m_old = m_ref[h]
            m_new = jnp.maximum(m_old, jnp.max(s_h, axis=1, keepdims=True))
            alpha = jnp.exp(m_old - m_new)
            p_h = jnp.exp(s_h - m_new) * crep
            l_ref[h] = l_ref[h] * alpha + jnp.sum(p_h, axis=1, keepdims=True)
            o_h = lax.dot_general(
                p_h, v_h, (((1,), (0,)), ((), ())),
                preferred_element_type=jnp.float32,
            )
            acc_ref[h] = acc_ref[h] * alpha + o_h
            m_ref[h] = m_new

        @pl.when(blk == NBLK - 1)
        def _():
            sbuf[:, :, :D] = acc_ref[...]
            sbuf[:, :, D:D + 1] = m_ref[...]
            sbuf[:, :, D + 1:D + 2] = l_ref[...]
            peers = [
                (my_x, my_y, 1 - my_z),
                (my_x, 1 - my_y, my_z),
                (1 - my_x, my_y, my_z),
            ]
            for stage, peer in enumerate(peers):
                rd = pltpu.make_async_remote_copy(
                    src_ref=sbuf, dst_ref=rbuf.at[stage],
                    send_sem=send_sems.at[stage],
                    recv_sem=recv_sems.at[stage],
                    device_id=peer, device_id_type=pl.DeviceIdType.MESH,
                )
                rd.start()
                rd.wait()

                m_o = rbuf[stage, :, :, D:D + 1]
                l_o = rbuf[stage, :, :, D + 1:D + 2]
                acc_o = rbuf[stage, :, :, :D]
                m_g = jnp.maximum(m_ref[...], m_o)
                a = jnp.exp(m_ref[...] - m_g)
                b = jnp.exp(m_o - m_g)
                l_ref[...] = l_ref[...] * a + l_o * b
                acc_ref[...] = acc_ref[...] * a + acc_o * b
                m_ref[...] = m_g
                if stage < 2:
                    sbuf[:, :, :D] = acc_ref[...]
                    sbuf[:, :, D:D + 1] = m_ref[...]
                    sbuf[:, :, D + 1:D + 2] = l_ref[...]

            out_ref[...] = acc_ref[...] / l_ref[...]

    out = pl.pallas_call(
        body,
        grid=(NBLK,),
        in_specs=[
            pl.BlockSpec((B, H * D), lambda b: (0, 0)),
            pl.BlockSpec(memory_space=pltpu.MemorySpace.HBM),
            pl.BlockSpec(memory_space=pltpu.MemorySpace.HBM),
            pl.BlockSpec((B, NB), lambda b: (0, 0)),
            pl.BlockSpec((B, 1), lambda b: (0, 0)),
        ],
        out_specs=pl.BlockSpec((H, B, D), lambda b: (0, 0, 0)),
        out_shape=jax.ShapeDtypeStruct((H, B, D), jnp.float32),
        scratch_shapes=[
            pltpu.VMEM((H, B, 1), jnp.float32),
            pltpu.VMEM((H, B, 1), jnp.float32),
            pltpu.VMEM((H, B, D), jnp.float32),
            pltpu.VMEM((2, H, KBLK, D), jnp.float32),
            pltpu.VMEM((2, H, KBLK, D), jnp.float32),
            pltpu.VMEM((H, B, 2 * D), jnp.float32),
            pltpu.VMEM((3, H, B, 2 * D), jnp.float32),
            pltpu.SemaphoreType.DMA((2,)),
            pltpu.SemaphoreType.DMA((2,)),
            pltpu.SemaphoreType.DMA((3,)),
            pltpu.SemaphoreType.DMA((3,)),
        ],
        compiler_params=pltpu.CompilerParams(
            dimension_semantics=("arbitrary",),
            collective_id=0,
        ),
    )(Q2, K3, V3, bt, lens2)
    return out.transpose(1, 0, 2).reshape(B, 1, H, D)
